# baseline (speedup 1.0000x reference)
import jax
import jax.numpy as jnp
from jax.experimental import pallas as pl
from jax.experimental.pallas import tpu as pltpu

N = 2048
D = 16
B = 8
M = 1152
MAX_ITER = 15
TOL = 0.001
BIGSLOT = 4096
BIGF = 3000.0

def _fold16(planes):
    for w in (8, 4, 2, 1):
        planes = [planes[i] + planes[i + w] for i in range(w)]
    return planes[0]


def _rowsum16(x):
    for w in (8, 4, 2, 1):
        x = x[:, :w] + x[:, w:2 * w]
    return x


def _kmeans_kernel(meta_ref, pos_ref, pos_t_ref, batch_row_ref, batch_col_ref,
                   out_ref, cents_ref, cls_ref):
    f32 = jnp.float32
    pos = pos_ref[...]
    pos_t = pos_t_ref[...]
    batch_row = batch_row_ref[...]
    batch_col = batch_col_ref[...]

    lane_n = jax.lax.broadcasted_iota(jnp.int32, (B, N), 1)
    lane_m8 = jax.lax.broadcasted_iota(jnp.int32, (B, M), 1)
    seg_mask = batch_row == jax.lax.broadcasted_iota(jnp.int32, (B, N), 0)

    pos_hi = pos.astype(jnp.bfloat16).astype(f32)
    pos_mid = (pos - pos_hi).astype(jnp.bfloat16).astype(f32)
    pos_lo = pos - pos_hi - pos_mid
    pos3 = jnp.concatenate([pos_hi, pos_mid, pos_lo], axis=1)

    def dist8(pts):
        chunks = []
        for c0 in range(0, N, 512):
            planes = []
            for dd in range(D):
                diff = pos_t[dd:dd + 1, c0:c0 + 512] - pts[:, dd:dd + 1]
                planes.append(diff * diff)
            chunks.append(jnp.sqrt(_fold16(planes)))
        return jnp.concatenate(chunks, axis=1)

    def split3(x):
        hi = x.astype(jnp.bfloat16).astype(f32)
        mid = (x - hi).astype(jnp.bfloat16).astype(f32)
        lo = x - hi - mid
        return jnp.concatenate([hi, mid, lo], axis=1)

    def gather8(idx_vec):
        oh = (lane_n == idx_vec).astype(f32)
        g3 = jax.lax.dot_general(oh, pos3, (((1,), (0,)), ((), ())))
        return (g3[:, :D] + g3[:, D:2 * D]) + g3[:, 2 * D:]

    def scatter_t(slot_vec, rows):
        oh = (lane_m8 == slot_vec).astype(f32)
        s3 = jax.lax.dot_general(split3(rows), oh, (((0,), (0,)), ((), ())))
        return (s3[:D, :] + s3[D:2 * D, :]) + s3[2 * D:, :]

    cents_ref[...] = jnp.zeros((D, M), f32)
    m_vec = jnp.concatenate(
        [jnp.full((1, 1), meta_ref[0, 1 + b], jnp.int32) for b in range(B)],
        axis=0)
    off_vec = jnp.concatenate(
        [jnp.full((1, 1), meta_ref[0, 9 + b], jnp.int32) for b in range(B)],
        axis=0)
    idx0 = jnp.min(jnp.where(seg_mask, lane_n, N), axis=1, keepdims=True)
    p0 = gather8(idx0)
    d = jnp.where(seg_mask, dist8(p0), -jnp.inf)
    slot0 = jnp.where(m_vec > 0, off_vec, BIGSLOT)
    cents_ref[...] += scatter_t(slot0, p0)

    steps = meta_ref[0, 0]

    def fps_body(t, d):
        maxv = jnp.max(d, axis=1, keepdims=True)
        j = jnp.min(jnp.where(d == maxv, lane_n, N), axis=1, keepdims=True)
        pj = gather8(j)
        d = jnp.where(seg_mask, jnp.minimum(d, dist8(pj)), -jnp.inf)
        slot = jnp.where(t < m_vec, off_vec + t, BIGSLOT)
        cents_ref[...] += scatter_t(slot, pj)
        return d

    jax.lax.fori_loop(1, steps + 1, fps_body, d)

    lane_m1 = jax.lax.broadcasted_iota(jnp.int32, (1, M), 1)
    cb_row = jnp.full((1, M), -1, jnp.int32)
    for b in range(B):
        off_b = meta_ref[0, 9 + b]
        m_b = meta_ref[0, 1 + b]
        inr = jnp.logical_and(lane_m1 >= off_b, lane_m1 < off_b + m_b)
        cb_row = jnp.where(inr, b, cb_row)

    p2_col = _rowsum16(pos * pos)
    ones_1n = jnp.ones((1, N), f32)
    lane_mn = jax.lax.broadcasted_iota(jnp.int32, (N, M), 1)
    bad = batch_col != cb_row

    def _colsum16(x):
        for w in (8, 4, 2, 1):
            x = x[:w, :] + x[w:2 * w, :]
        return x

    def km_cond(state):
        i, conv = state
        return jnp.logical_and(i < MAX_ITER, jnp.logical_not(conv))

    def km_body(state):
        i, _ = state
        cents_t = cents_ref[...]
        c2_row = _colsum16(cents_t * cents_t)
        prod = jax.lax.dot_general(pos, cents_t, (((1,), (0,)), ((), ())))
        dmat = p2_col + c2_row - 2.0 * prod
        dmat = jnp.where(bad, jnp.inf, dmat)
        minv = jnp.min(dmat, axis=1, keepdims=True)
        cls = jnp.min(jnp.where(dmat == minv, lane_mn, M), axis=1,
                      keepdims=True)
        oh = (lane_mn == cls).astype(f32)
        s3 = jax.lax.dot_general(pos3, oh, (((0,), (0,)), ((), ())))
        sums_t = (s3[:D, :] + s3[D:2 * D, :]) + s3[2 * D:, :]
        counts = jax.lax.dot_general(ones_1n, oh, (((1,), (0,)), ((), ())))
        mean_t = sums_t / jnp.maximum(counts, 1.0)
        cmask = (counts > 0.0).astype(f32)
        newc_t = mean_t * cmask + cents_t * (1.0 - cmask)
        disp = jnp.sqrt(_colsum16((cents_t - newc_t) ** 2))
        conv = jnp.all(disp < TOL)
        cents_ref[...] = newc_t
        cls_ref[...] = cls
        return i + 1, conv

    jax.lax.while_loop(km_cond, km_body,
                       (jnp.int32(0), jnp.bool_(False)))

    cls = cls_ref[...]
    ohb = lane_mn == cls
    idx_col = jax.lax.broadcasted_iota(jnp.int32, (N, 1), 0).astype(f32)
    cmin_row = jnp.min(jnp.where(ohb, idx_col, BIGF), axis=0,
                       keepdims=True)
    cmin_hi = cmin_row.astype(jnp.bfloat16).astype(f32)
    cmin_lo = cmin_row - cmin_hi
    cmin2 = jnp.concatenate([cmin_hi, cmin_lo], axis=0)
    rep2 = jax.lax.dot_general(ohb.astype(f32), cmin2,
                               (((1,), (1,)), ((), ())))
    rep = rep2[:, 0:1] + rep2[:, 1:2]
    lt = jnp.logical_and(cmin_row < rep, cmin_row < BIGF)
    labels = jax.lax.dot_general(lt.astype(f32), jnp.ones((M, 1), f32),
                                 (((1,), (0,)), ((), ())))
    out_ref[...] = labels.astype(jnp.int32)


def kernel(pos, batch):
    pos = pos.astype(jnp.float32)
    batch = batch.astype(jnp.int32)
    seg = jnp.arange(B, dtype=jnp.int32)
    n = jnp.sum(batch[None, :] == seg[:, None], axis=1).astype(jnp.int32)
    m = (n + 1) // 2
    off = jnp.cumsum(m) - m
    steps = jnp.maximum(jnp.max(m) - 1, 0)
    meta = jnp.concatenate([steps[None], m, off]).astype(jnp.int32)[None, :]

    out = pl.pallas_call(
        _kmeans_kernel,
        out_shape=jax.ShapeDtypeStruct((N, 1), jnp.int32),
        in_specs=[
            pl.BlockSpec(memory_space=pltpu.SMEM),
            pl.BlockSpec(memory_space=pltpu.VMEM),
            pl.BlockSpec(memory_space=pltpu.VMEM),
            pl.BlockSpec(memory_space=pltpu.VMEM),
            pl.BlockSpec(memory_space=pltpu.VMEM),
        ],
        scratch_shapes=[
            pltpu.VMEM((D, M), jnp.float32),
            pltpu.VMEM((N, 1), jnp.int32),
        ],
    )(meta, pos, pos.T, batch[None, :], batch[:, None])
    return out[:, 0]


# speedup vs baseline: 1.1232x; 1.1232x over previous
import jax
import jax.numpy as jnp
from jax.experimental import pallas as pl
from jax.experimental.pallas import tpu as pltpu

N = 2048
D = 16
B = 8
M = 1152
MAX_ITER = 15
TOL = 0.001
BIGSLOT = 4096
BIGF = 3000.0

def _fold16(planes):
    for w in (8, 4, 2, 1):
        planes = [planes[i] + planes[i + w] for i in range(w)]
    return planes[0]


def _rowsum16(x):
    for w in (8, 4, 2, 1):
        x = x[:, :w] + x[:, w:2 * w]
    return x


def _kmeans_kernel(meta_ref, pos_ref, pos_t_ref, batch_row_ref, batch_col_ref,
                   out_ref, cents_ref, cls_ref):
    f32 = jnp.float32
    pos = pos_ref[...]
    pos_t = pos_t_ref[...]
    batch_row = batch_row_ref[...]
    batch_col = batch_col_ref[...]

    lane_n = jax.lax.broadcasted_iota(jnp.int32, (B, N), 1)
    lane_m8 = jax.lax.broadcasted_iota(jnp.int32, (B, M), 1)
    seg_mask = batch_row == jax.lax.broadcasted_iota(jnp.int32, (B, N), 0)

    pos_hi = pos.astype(jnp.bfloat16).astype(f32)
    pos_mid = (pos - pos_hi).astype(jnp.bfloat16).astype(f32)
    pos_lo = pos - pos_hi - pos_mid
    pos3 = jnp.concatenate([pos_hi, pos_mid, pos_lo], axis=1)

    def dist8(pts):
        chunks = []
        for c0 in range(0, N, 512):
            planes = []
            for dd in range(D):
                diff = pos_t[dd:dd + 1, c0:c0 + 512] - pts[:, dd:dd + 1]
                planes.append(diff * diff)
            chunks.append(jnp.sqrt(_fold16(planes)))
        return jnp.concatenate(chunks, axis=1)

    def gather8(idx_vec):
        oh = (lane_n == idx_vec).astype(f32)
        g3 = jax.lax.dot_general(oh, pos3, (((1,), (0,)), ((), ())))
        return (g3[:, :D] + g3[:, D:2 * D]) + g3[:, 2 * D:]

    def slot_put(acc, slot_vec, idx_vec):
        return acc + jnp.where(lane_m8 == slot_vec, idx_vec, 0)

    m_vec = jnp.concatenate(
        [jnp.full((1, 1), meta_ref[0, 1 + b], jnp.int32) for b in range(B)],
        axis=0)
    off_vec = jnp.concatenate(
        [jnp.full((1, 1), meta_ref[0, 9 + b], jnp.int32) for b in range(B)],
        axis=0)
    idx0 = jnp.min(jnp.where(seg_mask, lane_n, N), axis=1, keepdims=True)
    p0 = gather8(idx0)
    d = jnp.where(seg_mask, dist8(p0), -jnp.inf)
    slot0 = jnp.where(m_vec > 0, off_vec, BIGSLOT)
    acc0 = slot_put(jnp.zeros((B, M), jnp.int32), slot0, idx0)

    steps = meta_ref[0, 0]

    def fps_body(t, state):
        d, acc = state
        maxv = jnp.max(d, axis=1, keepdims=True)
        j = jnp.min(jnp.where(d == maxv, lane_n, N), axis=1, keepdims=True)
        pj = gather8(j)
        d = jnp.where(seg_mask, jnp.minimum(d, dist8(pj)), -jnp.inf)
        slot = jnp.where(t < m_vec, off_vec + t, BIGSLOT)
        return d, slot_put(acc, slot, j)

    _, acc = jax.lax.fori_loop(1, steps + 1, fps_body, (d, acc0))

    fps_row = acc[0:1, :]
    for b in range(1, B):
        fps_row = fps_row + acc[b:b + 1, :]
    sub_n = jax.lax.broadcasted_iota(jnp.int32, (N, M), 0)
    ohg = (sub_n == fps_row).astype(f32)
    g3 = jax.lax.dot_general(pos3, ohg, (((0,), (0,)), ((), ())))
    cents_ref[...] = (g3[:D, :] + g3[D:2 * D, :]) + g3[2 * D:, :]

    lane_m1 = jax.lax.broadcasted_iota(jnp.int32, (1, M), 1)
    cb_row = jnp.full((1, M), -1, jnp.int32)
    for b in range(B):
        off_b = meta_ref[0, 9 + b]
        m_b = meta_ref[0, 1 + b]
        inr = jnp.logical_and(lane_m1 >= off_b, lane_m1 < off_b + m_b)
        cb_row = jnp.where(inr, b, cb_row)

    p2_col = _rowsum16(pos * pos)
    ones_1n = jnp.ones((1, N), f32)
    lane_mn = jax.lax.broadcasted_iota(jnp.int32, (N, M), 1)
    bad = batch_col != cb_row

    def _colsum16(x):
        for w in (8, 4, 2, 1):
            x = x[:w, :] + x[w:2 * w, :]
        return x

    def km_cond(state):
        i, conv = state
        return jnp.logical_and(i < MAX_ITER, jnp.logical_not(conv))

    def km_body(state):
        i, _ = state
        cents_t = cents_ref[...]
        c2_row = _colsum16(cents_t * cents_t)
        prod = jax.lax.dot_general(pos, cents_t, (((1,), (0,)), ((), ())))
        dmat = p2_col + c2_row - 2.0 * prod
        dmat = jnp.where(bad, jnp.inf, dmat)
        minv = jnp.min(dmat, axis=1, keepdims=True)
        cls = jnp.min(jnp.where(dmat == minv, lane_mn, M), axis=1,
                      keepdims=True)
        oh = (lane_mn == cls).astype(f32)
        s3 = jax.lax.dot_general(pos3, oh, (((0,), (0,)), ((), ())))
        sums_t = (s3[:D, :] + s3[D:2 * D, :]) + s3[2 * D:, :]
        counts = jax.lax.dot_general(ones_1n, oh, (((1,), (0,)), ((), ())))
        mean_t = sums_t / jnp.maximum(counts, 1.0)
        cmask = (counts > 0.0).astype(f32)
        newc_t = mean_t * cmask + cents_t * (1.0 - cmask)
        disp = jnp.sqrt(_colsum16((cents_t - newc_t) ** 2))
        conv = jnp.all(disp < TOL)
        cents_ref[...] = newc_t
        cls_ref[...] = cls
        return i + 1, conv

    jax.lax.while_loop(km_cond, km_body,
                       (jnp.int32(0), jnp.bool_(False)))

    cls = cls_ref[...]
    ohb = lane_mn == cls
    idx_col = jax.lax.broadcasted_iota(jnp.int32, (N, 1), 0).astype(f32)
    cmin_row = jnp.min(jnp.where(ohb, idx_col, BIGF), axis=0,
                       keepdims=True)
    cmin_hi = cmin_row.astype(jnp.bfloat16).astype(f32)
    cmin_lo = cmin_row - cmin_hi
    cmin2 = jnp.concatenate([cmin_hi, cmin_lo], axis=0)
    rep2 = jax.lax.dot_general(ohb.astype(f32), cmin2,
                               (((1,), (1,)), ((), ())))
    rep = rep2[:, 0:1] + rep2[:, 1:2]
    lt = jnp.logical_and(cmin_row < rep, cmin_row < BIGF)
    labels = jax.lax.dot_general(lt.astype(f32), jnp.ones((M, 1), f32),
                                 (((1,), (0,)), ((), ())))
    out_ref[...] = labels.astype(jnp.int32)


def kernel(pos, batch):
    pos = pos.astype(jnp.float32)
    batch = batch.astype(jnp.int32)
    seg = jnp.arange(B, dtype=jnp.int32)
    n = jnp.sum(batch[None, :] == seg[:, None], axis=1).astype(jnp.int32)
    m = (n + 1) // 2
    off = jnp.cumsum(m) - m
    steps = jnp.maximum(jnp.max(m) - 1, 0)
    meta = jnp.concatenate([steps[None], m, off]).astype(jnp.int32)[None, :]

    out = pl.pallas_call(
        _kmeans_kernel,
        out_shape=jax.ShapeDtypeStruct((N, 1), jnp.int32),
        in_specs=[
            pl.BlockSpec(memory_space=pltpu.SMEM),
            pl.BlockSpec(memory_space=pltpu.VMEM),
            pl.BlockSpec(memory_space=pltpu.VMEM),
            pl.BlockSpec(memory_space=pltpu.VMEM),
            pl.BlockSpec(memory_space=pltpu.VMEM),
        ],
        scratch_shapes=[
            pltpu.VMEM((D, M), jnp.float32),
            pltpu.VMEM((N, 1), jnp.int32),
        ],
    )(meta, pos, pos.T, batch[None, :], batch[:, None])
    return out[:, 0]
